# baseline (speedup 1.0000x reference)
import functools

import jax
import jax.numpy as jnp
from jax.experimental import pallas as pl
from jax.experimental.pallas import tpu as pltpu

PACK = 2


def _silu(v):
    t = 0.5 * v
    return t + t * jnp.tanh(t)


def _mbconv_kernel(x_ref, wbd_ref, b1_ref, wdd_ref, b2_ref,
                   wse1_ref, bse1_ref, wse2_ref, bse2_ref, wpbd_ref,
                   b3_ref, o_ref, halo_ref, *, K, H, W, LEFT):
    pad = (K - 1) // 2
    C2 = wbd_ref.shape[1]
    HW = H * W

    @pl.when(pl.program_id(0) == 0)
    def _zero_halo():
        halo_ref[...] = jnp.zeros_like(halo_ref)

    x = x_ref[...].reshape(x_ref.shape[0] * x_ref.shape[1], HW)

    y = jax.lax.dot_general(x.astype(jnp.bfloat16), wbd_ref[...],
                            (((0,), (0,)), ((), ())),
                            preferred_element_type=jnp.float32)
    y = _silu(y + b1_ref[...])

    halo_ref[pad:pad + H, LEFT:LEFT + W, :] = y.reshape(H, W, C2)

    acc = None
    for kh in range(K):
        for kw in range(K):
            col = LEFT - pad + kw
            t = halo_ref[kh:kh + H, col:col + W, :] * wdd_ref[kh, kw, :]
            acc = t if acc is None else acc + t
    z = _silu(acc + b2_ref[...])

    pooled = jnp.mean(z.reshape(HW, C2), axis=0, keepdims=True)
    h = jnp.dot(pooled, wse1_ref[...],
                preferred_element_type=jnp.float32) + bse1_ref[...]
    h = _silu(h)
    g = jnp.dot(h, wse2_ref[...],
                preferred_element_type=jnp.float32) + bse2_ref[...]
    se = 0.5 + 0.5 * jnp.tanh(0.5 * g)

    wp = (wpbd_ref[...] * jnp.transpose(se, (1, 0))).astype(jnp.bfloat16)
    ot = jax.lax.dot_general(wp, z.reshape(HW, C2).astype(jnp.bfloat16),
                             (((0,), (1,)), ((), ())),
                             preferred_element_type=jnp.float32)
    res = (ot + b3_ref[...] + x).astype(o_ref.dtype)
    o_ref[...] = res.reshape(o_ref.shape)


def _block_diag(w):
    return jnp.kron(jnp.eye(PACK, dtype=w.dtype), w)


def kernel(x, w_exp, s1, b1, w_dw, s2, b2, w_se1, b_se1, w_se2, b_se2,
           w_proj, s3, b3):
    N, Cin, H, W = x.shape
    Cexp = w_exp.shape[1]
    Cout = w_proj.shape[1]
    K = w_dw.shape[0]
    HW = H * W
    pad = (K - 1) // 2
    LEFT = max(8, 8 * pl.cdiv(pad, 8))
    Hp = H + 2 * pad
    Wp = LEFT + W + pad
    NP = N // PACK
    C2, CO2 = PACK * Cexp, PACK * Cout

    x_blk = x
    t2 = lambda v: jnp.tile(v, PACK).reshape(1, -1)
    wbd = (_block_diag(w_exp) * t2(s1)).astype(jnp.bfloat16)
    wse1bd = _block_diag(w_se1)
    wse2bd = _block_diag(w_se2)
    wpbd = _block_diag(w_proj) * t2(s3)
    wdd = jnp.tile(w_dw, (1, 1, PACK)) * t2(s2)
    Csq2 = wse1bd.shape[1]

    out = pl.pallas_call(
        functools.partial(_mbconv_kernel, K=K, H=H, W=W, LEFT=LEFT),
        out_shape=jax.ShapeDtypeStruct((N, Cout, H, W), x.dtype),
        grid=(NP,),
        in_specs=[
            pl.BlockSpec((PACK, Cin, H, W), lambda n: (n, 0, 0, 0)),
            pl.BlockSpec((PACK * Cin, C2), lambda n: (0, 0)),
            pl.BlockSpec((1, C2), lambda n: (0, 0)),
            pl.BlockSpec((K, K, C2), lambda n: (0, 0, 0)),
            pl.BlockSpec((1, C2), lambda n: (0, 0)),
            pl.BlockSpec((C2, Csq2), lambda n: (0, 0)),
            pl.BlockSpec((1, Csq2), lambda n: (0, 0)),
            pl.BlockSpec((Csq2, C2), lambda n: (0, 0)),
            pl.BlockSpec((1, C2), lambda n: (0, 0)),
            pl.BlockSpec((C2, CO2), lambda n: (0, 0)),
            pl.BlockSpec((CO2, 1), lambda n: (0, 0)),
        ],
        out_specs=pl.BlockSpec((PACK, Cout, H, W), lambda n: (n, 0, 0, 0)),
        scratch_shapes=[pltpu.VMEM((Hp, Wp, C2), jnp.float32)],
        compiler_params=pltpu.CompilerParams(
            dimension_semantics=("arbitrary",)),
    )(x_blk, wbd, t2(b1), wdd, t2(b2),
      wse1bd, t2(b_se1), wse2bd, t2(b_se2), wpbd,
      t2(b3).reshape(CO2, 1))
    return out


# speedup vs baseline: 1.0092x; 1.0092x over previous
import functools

import jax
import jax.numpy as jnp
from jax.experimental import pallas as pl
from jax.experimental.pallas import tpu as pltpu

PACK = 2


def _silu(v):
    t = 0.5 * v
    return t + t * jnp.tanh(t)


def _mbconv_kernel(x_ref, wbd_ref, b1_ref, wdd_ref, b2_ref,
                   wse1_ref, bse1_ref, wse2_ref, bse2_ref, wpbd_ref,
                   b3_ref, o_ref, halo_ref, *, K, H, W, LEFT, PAIRS):
    pad = (K - 1) // 2
    C2 = wbd_ref.shape[1]
    HW = H * W
    CI2 = x_ref.shape[1] * PACK

    @pl.when(pl.program_id(0) == 0)
    def _zero_halo():
        halo_ref[...] = jnp.zeros_like(halo_ref)

    xall = x_ref[...].reshape(PAIRS * CI2, HW)

    for q in range(PAIRS):
        x = xall[q * CI2:(q + 1) * CI2]

        y = jax.lax.dot_general(x.astype(jnp.bfloat16), wbd_ref[...],
                                (((0,), (0,)), ((), ())),
                                preferred_element_type=jnp.float32)
        y = _silu(y + b1_ref[...])

        halo_ref[q, pad:pad + H, LEFT:LEFT + W, :] = y.reshape(H, W, C2)

        acc = None
        for kh in range(K):
            for kw in range(K):
                col = LEFT - pad + kw
                t = (halo_ref[q, kh:kh + H, col:col + W, :]
                     * wdd_ref[kh, kw, :])
                acc = t if acc is None else acc + t
        z = _silu(acc + b2_ref[...])

        pooled = jnp.mean(z.reshape(HW, C2), axis=0, keepdims=True)
        h = jnp.dot(pooled, wse1_ref[...],
                    preferred_element_type=jnp.float32) + bse1_ref[...]
        h = _silu(h)
        g = jnp.dot(h, wse2_ref[...],
                    preferred_element_type=jnp.float32) + bse2_ref[...]
        se = 0.5 + 0.5 * jnp.tanh(0.5 * g)

        wp = (wpbd_ref[...]
              * jnp.transpose(se, (1, 0))).astype(jnp.bfloat16)
        ot = jax.lax.dot_general(wp, z.reshape(HW, C2).astype(jnp.bfloat16),
                                 (((0,), (1,)), ((), ())),
                                 preferred_element_type=jnp.float32)
        res = (ot + b3_ref[...] + x).astype(o_ref.dtype)
        o_ref[q * PACK:(q + 1) * PACK] = res.reshape(
            PACK, o_ref.shape[1], H, W)


def _block_diag(w):
    return jnp.kron(jnp.eye(PACK, dtype=w.dtype), w)


def kernel(x, w_exp, s1, b1, w_dw, s2, b2, w_se1, b_se1, w_se2, b_se2,
           w_proj, s3, b3):
    N, Cin, H, W = x.shape
    Cexp = w_exp.shape[1]
    Cout = w_proj.shape[1]
    K = w_dw.shape[0]
    HW = H * W
    pad = (K - 1) // 2
    LEFT = max(8, 8 * pl.cdiv(pad, 8))
    Hp = H + 2 * pad
    Wp = LEFT + W + pad
    PAIRS = 2
    NP = N // (PACK * PAIRS)
    C2, CO2 = PACK * Cexp, PACK * Cout

    x_blk = x
    t2 = lambda v: jnp.tile(v, PACK).reshape(1, -1)
    wbd = (_block_diag(w_exp) * t2(s1)).astype(jnp.bfloat16)
    wse1bd = _block_diag(w_se1)
    wse2bd = _block_diag(w_se2)
    wpbd = _block_diag(w_proj) * t2(s3)
    wdd = jnp.tile(w_dw, (1, 1, PACK)) * t2(s2)
    Csq2 = wse1bd.shape[1]

    out = pl.pallas_call(
        functools.partial(_mbconv_kernel, K=K, H=H, W=W, LEFT=LEFT,
                          PAIRS=PAIRS),
        out_shape=jax.ShapeDtypeStruct((N, Cout, H, W), x.dtype),
        grid=(NP,),
        in_specs=[
            pl.BlockSpec((PACK * PAIRS, Cin, H, W), lambda n: (n, 0, 0, 0)),
            pl.BlockSpec((PACK * Cin, C2), lambda n: (0, 0)),
            pl.BlockSpec((1, C2), lambda n: (0, 0)),
            pl.BlockSpec((K, K, C2), lambda n: (0, 0, 0)),
            pl.BlockSpec((1, C2), lambda n: (0, 0)),
            pl.BlockSpec((C2, Csq2), lambda n: (0, 0)),
            pl.BlockSpec((1, Csq2), lambda n: (0, 0)),
            pl.BlockSpec((Csq2, C2), lambda n: (0, 0)),
            pl.BlockSpec((1, C2), lambda n: (0, 0)),
            pl.BlockSpec((C2, CO2), lambda n: (0, 0)),
            pl.BlockSpec((CO2, 1), lambda n: (0, 0)),
        ],
        out_specs=pl.BlockSpec((PACK * PAIRS, Cout, H, W),
                               lambda n: (n, 0, 0, 0)),
        scratch_shapes=[pltpu.VMEM((PAIRS, Hp, Wp, C2), jnp.float32)],
        compiler_params=pltpu.CompilerParams(
            dimension_semantics=("arbitrary",)),
    )(x_blk, wbd, t2(b1), wdd, t2(b2),
      wse1bd, t2(b_se1), wse2bd, t2(b_se2), wpbd,
      t2(b3).reshape(CO2, 1))
    return out


# speedup vs baseline: 1.0201x; 1.0107x over previous
import functools

import jax
import jax.numpy as jnp
from jax.experimental import pallas as pl
from jax.experimental.pallas import tpu as pltpu

PACK = 2


def _silu(v):
    t = 0.5 * v
    return t + t * jnp.tanh(t)


def _mbconv_kernel(x_ref, wbd_ref, wf_ref, o_ref, halo_ref,
                   *, K, H, W, LEFT, PAIRS, CSQ2, CO2):
    pad = (K - 1) // 2
    C2 = wbd_ref.shape[1]
    HW = H * W
    CI2 = x_ref.shape[1] * PACK

    r_se1 = K * K
    r_se2 = r_se1 + CSQ2
    r_wp = r_se2 + CSQ2
    r_b = r_wp + CO2
    b1 = wf_ref[r_b:r_b + 1, :]
    b2 = wf_ref[r_b + 1:r_b + 2, :]
    bse1 = wf_ref[r_b + 2:r_b + 3, 0:CSQ2]
    bse2 = wf_ref[r_b + 3:r_b + 4, :]
    b3col = jnp.transpose(wf_ref[r_b + 4:r_b + 5, 0:CO2], (1, 0))

    @pl.when(pl.program_id(0) == 0)
    def _zero_halo():
        halo_ref[...] = jnp.zeros_like(halo_ref)

    xall = x_ref[...].reshape(PAIRS * CI2, HW)

    for q in range(PAIRS):
        x = xall[q * CI2:(q + 1) * CI2]

        y = jax.lax.dot_general(x.astype(jnp.bfloat16), wbd_ref[...],
                                (((0,), (0,)), ((), ())),
                                preferred_element_type=jnp.float32)
        y = _silu(y + b1)

        halo_ref[q, pad:pad + H, LEFT:LEFT + W, :] = y.reshape(H, W, C2)

        acc = None
        for kh in range(K):
            for kw in range(K):
                col = LEFT - pad + kw
                t = (halo_ref[q, kh:kh + H, col:col + W, :]
                     * wf_ref[kh * K + kw])
                acc = t if acc is None else acc + t
        z = _silu(acc + b2)

        pooled = jnp.mean(z.reshape(HW, C2), axis=0, keepdims=True)
        h = jax.lax.dot_general(pooled, wf_ref[r_se1:r_se1 + CSQ2, :],
                                (((1,), (1,)), ((), ())),
                                preferred_element_type=jnp.float32) + bse1
        h = _silu(h)
        g = jax.lax.dot_general(h, wf_ref[r_se2:r_se2 + CSQ2, :],
                                (((1,), (0,)), ((), ())),
                                preferred_element_type=jnp.float32) + bse2
        se = 0.5 + 0.5 * jnp.tanh(0.5 * g)

        wp = (wf_ref[r_wp:r_wp + CO2, :] * se).astype(jnp.bfloat16)
        ot = jax.lax.dot_general(wp, z.reshape(HW, C2).astype(jnp.bfloat16),
                                 (((1,), (1,)), ((), ())),
                                 preferred_element_type=jnp.float32)
        res = (ot + b3col + x).astype(o_ref.dtype)
        o_ref[q * PACK:(q + 1) * PACK] = res.reshape(
            PACK, o_ref.shape[1], H, W)


def _block_diag(w):
    return jnp.kron(jnp.eye(PACK, dtype=w.dtype), w)


def kernel(x, w_exp, s1, b1, w_dw, s2, b2, w_se1, b_se1, w_se2, b_se2,
           w_proj, s3, b3):
    N, Cin, H, W = x.shape
    Cexp = w_exp.shape[1]
    Cout = w_proj.shape[1]
    K = w_dw.shape[0]
    HW = H * W
    pad = (K - 1) // 2
    LEFT = max(8, 8 * pl.cdiv(pad, 8))
    Hp = H + 2 * pad
    Wp = LEFT + W + pad
    PAIRS = 2
    NP = N // (PACK * PAIRS)
    C2, CO2 = PACK * Cexp, PACK * Cout

    t2 = lambda v: jnp.tile(v, PACK).reshape(1, -1)
    rowpad = lambda r: jnp.concatenate(
        [r, jnp.zeros((1, C2 - r.shape[1]), r.dtype)], axis=1)
    Csq2 = PACK * w_se1.shape[1]
    wbd = (_block_diag(w_exp) * t2(s1)).astype(jnp.bfloat16)
    rows = [
        jnp.tile(w_dw.reshape(K * K, Cexp), (1, PACK)) * t2(s2),
        _block_diag(w_se1).T,
        _block_diag(w_se2),
        (_block_diag(w_proj) * t2(s3)).T,
        t2(b1), t2(b2), rowpad(t2(b_se1)), t2(b_se2), rowpad(t2(b3)),
    ]
    wf = jnp.concatenate(rows, axis=0)
    nrows = wf.shape[0]
    if nrows % 8:
        wf = jnp.concatenate(
            [wf, jnp.zeros((8 - nrows % 8, C2), wf.dtype)], axis=0)

    out = pl.pallas_call(
        functools.partial(_mbconv_kernel, K=K, H=H, W=W, LEFT=LEFT,
                          PAIRS=PAIRS, CSQ2=Csq2, CO2=CO2),
        out_shape=jax.ShapeDtypeStruct((N, Cout, H, W), x.dtype),
        grid=(NP,),
        in_specs=[
            pl.BlockSpec((PACK * PAIRS, Cin, H, W), lambda n: (n, 0, 0, 0)),
            pl.BlockSpec((PACK * Cin, C2), lambda n: (0, 0)),
            pl.BlockSpec(wf.shape, lambda n: (0, 0)),
        ],
        out_specs=pl.BlockSpec((PACK * PAIRS, Cout, H, W),
                               lambda n: (n, 0, 0, 0)),
        scratch_shapes=[pltpu.VMEM((PAIRS, Hp, Wp, C2), jnp.float32)],
        compiler_params=pltpu.CompilerParams(
            dimension_semantics=("arbitrary",)),
    )(x, wbd, wf)
    return out
